# baseline (speedup 1.0000x reference)
import functools

import jax
import jax.numpy as jnp
from jax.experimental import pallas as pl
from jax.experimental.pallas import tpu as pltpu

_NUM_LUT = 3
_NV = 128
_OUT_F = 2048
_LUT = 16
_VEC = 16
_VPG = 8
_NG = 16
_R = _OUT_F // 8


def _body(gate_ref, bd_ref, sc_ref, zr_ref, out_ref):
    seg = jax.lax.broadcasted_iota(jnp.int32, (_R, 128), 1) % _LUT
    inv_seg = 15 - seg
    low4 = jnp.int32(15)
    neg = jnp.float32(-3.0e38)
    masks_f = [seg < _LUT - s for s in (1, 2, 4, 8)]
    masks_b = [seg >= s for s in (1, 2, 4, 8)]

    ai = jax.lax.broadcasted_iota(jnp.int32, (_VPG, 128), 0)
    cj = jax.lax.broadcasted_iota(jnp.int32, (_VPG, 128), 1)
    e8 = jnp.where(cj // _LUT == ai, 1.0, 0.0).astype(jnp.float32)
    s128 = jax.lax.dot(sc_ref[0], e8, precision=jax.lax.Precision.HIGHEST)
    z128 = jax.lax.dot(zr_ref[0], e8, precision=jax.lax.Precision.HIGHEST)

    for vv in range(8):
        ohs = []
        for l in range(_NUM_LUT):
            gi = gate_ref[l, vv]
            x = jax.lax.bitcast_convert_type(
                (jax.lax.bitcast_convert_type(gi, jnp.int32) & ~low4) | inv_seg,
                jnp.float32)
            for i, s in enumerate((1, 2, 4, 8)):
                yf = pltpu.roll(x, 128 - s, 1)
                x = jnp.maximum(x, jnp.where(masks_f[i], yf, neg))
                yb = pltpu.roll(x, s, 1)
                x = jnp.maximum(x, jnp.where(masks_b[i], yb, neg))
            xi = jax.lax.bitcast_convert_type(x, jnp.int32) & low4
            ohs.append((xi == inv_seg).astype(jnp.float32))
        oh = jnp.concatenate(ohs, axis=1)
        w = jax.lax.dot(oh, bd_ref[vv])
        out_ref[vv] = (w - z128) * s128


@jax.jit
def kernel(gate, codebook, scales, zeros):
    gv = gate.reshape(_NUM_LUT, _NV, _R, 128)
    st = scales.T.reshape(_NG, _R, _VPG)
    zt = zeros.astype(jnp.float32).T.reshape(_NG, _R, _VPG)
    eye8 = jnp.eye(_VPG, dtype=codebook.dtype)
    bd = jnp.einsum("lvkj,ab->vlakbj", codebook, eye8)
    bd = bd.reshape(_NV, _NUM_LUT * _VPG * _LUT, _VPG * _VEC)

    res = pl.pallas_call(
        _body,
        grid=(_NG,),
        in_specs=[
            pl.BlockSpec((_NUM_LUT, _VPG, _R, 128), lambda g: (0, g, 0, 0)),
            pl.BlockSpec((_VPG, _NUM_LUT * _VPG * _LUT, _VPG * _VEC),
                         lambda g: (g, 0, 0)),
            pl.BlockSpec((1, _R, _VPG), lambda g: (g, 0, 0)),
            pl.BlockSpec((1, _R, _VPG), lambda g: (g, 0, 0)),
        ],
        out_specs=pl.BlockSpec((_VPG, _R, 128), lambda g: (g, 0, 0)),
        out_shape=jax.ShapeDtypeStruct((_NV, _R, 128), jnp.float32),
    )(gv, bd, st, zt)

    return res.reshape(_NV, _OUT_F, _VEC).transpose(1, 0, 2).reshape(_OUT_F, _NV * _VEC)


# speedup vs baseline: 1.3579x; 1.3579x over previous
import functools

import jax
import jax.numpy as jnp
from jax.experimental import pallas as pl
from jax.experimental.pallas import tpu as pltpu

_NUM_LUT = 3
_NV = 128
_OUT_F = 2048
_LUT = 16
_VEC = 16
_VPG = 8
_NG = 16
_R = _OUT_F // 8


def _body(gate_ref, bd_ref, sc_ref, zr_ref, out_ref):
    lane = jax.lax.broadcasted_iota(jnp.int32, (_R, 128), 1)
    seg = lane % _LUT
    inv16 = (15 - seg).astype(jnp.int16)
    low4 = jnp.int16(15)
    bmasks = [(lane & s) != 0 for s in (1, 2, 4, 8)]
    one_bf = jnp.bfloat16(1.0)
    zero_bf = jnp.bfloat16(0.0)

    ai = jax.lax.broadcasted_iota(jnp.int32, (_VPG, 128), 0)
    cj = jax.lax.broadcasted_iota(jnp.int32, (_VPG, 128), 1)
    e8 = jnp.where(cj // _LUT == ai, 1.0, 0.0).astype(jnp.float32)
    s128 = jax.lax.dot(sc_ref[0], e8, precision=jax.lax.Precision.HIGHEST)
    z128 = jax.lax.dot(zr_ref[0], e8, precision=jax.lax.Precision.HIGHEST)

    for vv in range(8):
        ohs = []
        for l in range(_NUM_LUT):
            gi = gate_ref[l, vv]
            xi = jax.lax.bitcast_convert_type(gi.astype(jnp.bfloat16), jnp.int16)
            x = jax.lax.bitcast_convert_type((xi & ~low4) | inv16, jnp.bfloat16)
            for i, s in enumerate((1, 2, 4, 8)):
                pf = pltpu.roll(x, 128 - s, 1)
                pb = pltpu.roll(x, s, 1)
                x = jnp.maximum(x, jnp.where(bmasks[i], pb, pf))
            wi = jax.lax.bitcast_convert_type(x, jnp.int16) & low4
            ohs.append(jnp.where(wi == inv16, one_bf, zero_bf))
        oh = jnp.concatenate(ohs, axis=1)
        w = jax.lax.dot(oh, bd_ref[vv],
                        preferred_element_type=jnp.float32)
        out_ref[vv] = (w - z128) * s128


@jax.jit
def kernel(gate, codebook, scales, zeros):
    gv = gate.reshape(_NUM_LUT, _NV, _R, 128)
    st = scales.T.reshape(_NG, _R, _VPG)
    zt = zeros.astype(jnp.float32).T.reshape(_NG, _R, _VPG)
    eye8 = jnp.eye(_VPG, dtype=codebook.dtype)
    bd = jnp.einsum("lvkj,ab->vlakbj", codebook, eye8)
    bd = bd.reshape(_NV, _NUM_LUT * _VPG * _LUT, _VPG * _VEC).astype(jnp.bfloat16)

    res = pl.pallas_call(
        _body,
        grid=(_NG,),
        in_specs=[
            pl.BlockSpec((_NUM_LUT, _VPG, _R, 128), lambda g: (0, g, 0, 0)),
            pl.BlockSpec((_VPG, _NUM_LUT * _VPG * _LUT, _VPG * _VEC),
                         lambda g: (g, 0, 0)),
            pl.BlockSpec((1, _R, _VPG), lambda g: (g, 0, 0)),
            pl.BlockSpec((1, _R, _VPG), lambda g: (g, 0, 0)),
        ],
        out_specs=pl.BlockSpec((_VPG, _R, 128), lambda g: (g, 0, 0)),
        out_shape=jax.ShapeDtypeStruct((_NV, _R, 128), jnp.float32),
    )(gv, bd, st, zt)

    return res.reshape(_NV, _OUT_F, _VEC).transpose(1, 0, 2).reshape(_OUT_F, _NV * _VEC)


# speedup vs baseline: 1.7192x; 1.2661x over previous
import functools

import jax
import jax.numpy as jnp
from jax.experimental import pallas as pl
from jax.experimental.pallas import tpu as pltpu

_NUM_LUT = 3
_NV = 128
_OUT_F = 2048
_LUT = 16
_VEC = 16
_VPG = 8
_NG = 16
_R = _OUT_F // 8


def _body(gate_ref, bd_ref, sc_ref, zr_ref, out_ref):
    lane = jax.lax.broadcasted_iota(jnp.int32, (_R, 128), 1)
    seg = lane % _LUT
    inv16 = (15 - seg).astype(jnp.int16)
    low4 = jnp.int16(15)
    smasks = [seg < _LUT - s for s in (1, 2, 4, 8)]
    neg = jnp.bfloat16(-3.0e38)
    one_bf = jnp.bfloat16(1.0)
    zero_bf = jnp.bfloat16(0.0)
    l3 = jax.lax.broadcasted_iota(jnp.int32, (384, 384), 0)
    c3 = jax.lax.broadcasted_iota(jnp.int32, (384, 384), 1)
    e3 = jnp.where((l3 % _LUT == 0) & (l3 // _LUT == c3 // _LUT),
                   1.0, 0.0).astype(jnp.bfloat16)
    inv48 = jnp.concatenate([15 - seg] * _NUM_LUT, axis=1)

    ai = jax.lax.broadcasted_iota(jnp.int32, (_VPG, 128), 0)
    cj = jax.lax.broadcasted_iota(jnp.int32, (_VPG, 128), 1)
    e8 = jnp.where(cj // _LUT == ai, 1.0, 0.0).astype(jnp.float32)
    s128 = jax.lax.dot(sc_ref[0], e8, precision=jax.lax.Precision.HIGHEST)
    z128 = jax.lax.dot(zr_ref[0], e8, precision=jax.lax.Precision.HIGHEST)

    for vv in range(8):
        xs = []
        for l in range(_NUM_LUT):
            gi = gate_ref[l, vv]
            xi = jax.lax.bitcast_convert_type(gi.astype(jnp.bfloat16), jnp.int16)
            x = jax.lax.bitcast_convert_type((xi & ~low4) | inv16, jnp.bfloat16)
            for i, s in enumerate((1, 2, 4, 8)):
                y = pltpu.roll(x, 128 - s, 1)
                x = jnp.maximum(x, jnp.where(smasks[i], y, neg))
            xs.append(x)
        x3 = jnp.concatenate(xs, axis=1)
        m3 = jax.lax.dot(x3, e3, preferred_element_type=jnp.float32)
        wi = (jax.lax.bitcast_convert_type(m3, jnp.int32) >> 16) & 15
        oh = jnp.where(wi == inv48, 1.0, 0.0).astype(jnp.bfloat16)
        w = jax.lax.dot(oh, bd_ref[vv],
                        preferred_element_type=jnp.float32)
        out_ref[vv] = (w - z128) * s128


@jax.jit
def kernel(gate, codebook, scales, zeros):
    gv = gate.reshape(_NUM_LUT, _NV, _R, 128)
    st = scales.T.reshape(_NG, _R, _VPG)
    zt = zeros.astype(jnp.float32).T.reshape(_NG, _R, _VPG)
    eye8 = jnp.eye(_VPG, dtype=codebook.dtype)
    bd = jnp.einsum("lvkj,ab->vlakbj", codebook, eye8)
    bd = bd.reshape(_NV, _NUM_LUT * _VPG * _LUT, _VPG * _VEC).astype(jnp.bfloat16)

    res = pl.pallas_call(
        _body,
        grid=(_NG,),
        in_specs=[
            pl.BlockSpec((_NUM_LUT, _VPG, _R, 128), lambda g: (0, g, 0, 0)),
            pl.BlockSpec((_VPG, _NUM_LUT * _VPG * _LUT, _VPG * _VEC),
                         lambda g: (g, 0, 0)),
            pl.BlockSpec((1, _R, _VPG), lambda g: (g, 0, 0)),
            pl.BlockSpec((1, _R, _VPG), lambda g: (g, 0, 0)),
        ],
        out_specs=pl.BlockSpec((_VPG, _R, 128), lambda g: (g, 0, 0)),
        out_shape=jax.ShapeDtypeStruct((_NV, _R, 128), jnp.float32),
    )(gv, bd, st, zt)

    return res.reshape(_NV, _OUT_F, _VEC).transpose(1, 0, 2).reshape(_OUT_F, _NV * _VEC)
